# Initial kernel scaffold; baseline (speedup 1.0000x reference)
import jax
import jax.numpy as jnp
from jax.experimental import pallas as pl


def kernel(input_ids, embedding_table):
    raise NotImplementedError("write your pallas kernel here")


# speedup vs baseline: 1.4704x; 1.4704x over previous
import functools

import jax
import jax.numpy as jnp
from jax import lax
from jax.experimental import pallas as pl
from jax.experimental.pallas import tpu as pltpu
from jax.experimental.pallas import tpu_sc as plsc

_VOCAB = 32002
_HIDDEN = 2048
_BATCH = 4
_SEQ = 4096
_NTOK = _BATCH * _SEQ
_NW = 32
_PER_W = _NTOK // _NW
_CHUNK = 16
_NCHUNK = _PER_W // _CHUNK

_mesh = plsc.VectorSubcoreMesh(core_axis_name="c", subcore_axis_name="s")


@functools.partial(
    pl.kernel,
    out_type=jax.ShapeDtypeStruct((_NTOK, _HIDDEN), jnp.float32),
    mesh=_mesh,
    scratch_types=[
        pltpu.VMEM((_NCHUNK, _CHUNK), jnp.int32),
        pltpu.VMEM((_CHUNK, _HIDDEN), jnp.float32),
        pltpu.SemaphoreType.DMA,
    ],
)
def _embed_lookup(table_hbm, idx_hbm, out_hbm, idx_v, buf, sem):
    wid = lax.axis_index("s") * 2 + lax.axis_index("c")
    base = wid * _PER_W
    pltpu.sync_copy(idx_hbm.at[wid], idx_v)

    def step(j, carry):
        pltpu.async_copy(table_hbm.at[idx_v.at[j]], buf, sem).wait()
        pltpu.sync_copy(buf, out_hbm.at[pl.ds(base + j * _CHUNK, _CHUNK)])
        return carry

    lax.fori_loop(0, _NCHUNK, step, 0)


def kernel(input_ids, embedding_table):
    ids = input_ids.reshape(_NW, _NCHUNK, _CHUNK)
    out = _embed_lookup(embedding_table, ids)
    return out.reshape(_BATCH, _SEQ, _HIDDEN)


# speedup vs baseline: 1.7224x; 1.1714x over previous
import functools

import jax
import jax.numpy as jnp
from jax import lax
from jax.experimental import pallas as pl
from jax.experimental.pallas import tpu as pltpu
from jax.experimental.pallas import tpu_sc as plsc

_VOCAB = 32002
_HIDDEN = 2048
_BATCH = 4
_SEQ = 4096
_NTOK = _BATCH * _SEQ
_NW = 32
_PER_W = _NTOK // _NW
_CHUNK = 16
_NCHUNK = _PER_W // _CHUNK

_mesh = plsc.VectorSubcoreMesh(core_axis_name="c", subcore_axis_name="s")


@functools.partial(
    pl.kernel,
    out_type=jax.ShapeDtypeStruct((_NTOK, _HIDDEN), jnp.float32),
    mesh=_mesh,
    scratch_types=[
        pltpu.VMEM((_NCHUNK, _CHUNK), jnp.int32),
        pltpu.VMEM((_CHUNK, _HIDDEN), jnp.float32),
        pltpu.VMEM((_CHUNK, _HIDDEN), jnp.float32),
        pltpu.SemaphoreType.DMA,
        pltpu.SemaphoreType.DMA,
        pltpu.SemaphoreType.DMA,
        pltpu.SemaphoreType.DMA,
    ],
)
def _embed_lookup(table_hbm, idx_hbm, out_hbm, idx_v,
                  buf0, buf1, g0, g1, o0, o1):
    wid = lax.axis_index("s") * 2 + lax.axis_index("c")
    base = wid * _PER_W
    pltpu.sync_copy(idx_hbm.at[wid], idx_v)

    bufs = (buf0, buf1)
    gsems = (g0, g1)
    osems = (o0, o1)

    def gather_start(j, b):
        pltpu.async_copy(table_hbm.at[idx_v.at[j]], bufs[b], gsems[b])

    def gather_wait(b):
        pltpu.make_async_copy(
            table_hbm.at[idx_v.at[0]], bufs[b], gsems[b]).wait()

    def out_start(j, b):
        pltpu.async_copy(
            bufs[b], out_hbm.at[pl.ds(base + j * _CHUNK, _CHUNK)], osems[b])

    def out_wait(b):
        pltpu.make_async_copy(
            bufs[b], out_hbm.at[pl.ds(base, _CHUNK)], osems[b]).wait()

    gather_start(0, 0)
    gather_start(1, 1)
    gather_wait(0)
    out_start(0, 0)
    gather_wait(1)
    out_start(1, 1)

    def step(k, carry):
        for b in range(2):
            j = 2 * k + b
            out_wait(b)
            gather_start(j, b)
            gather_wait(b)
            out_start(j, b)
        return carry

    lax.fori_loop(1, _NCHUNK // 2, step, 0)
    out_wait(0)
    out_wait(1)


def kernel(input_ids, embedding_table):
    ids = input_ids.reshape(_NW, _NCHUNK, _CHUNK)
    out = _embed_lookup(embedding_table, ids)
    return out.reshape(_BATCH, _SEQ, _HIDDEN)
